# baseline (speedup 1.0000x reference)
import jax
import jax.numpy as jnp
from jax import lax
from jax.experimental import pallas as pl
from jax.experimental.pallas import tpu as pltpu
from jax.experimental.pallas import tpu_sc as plsc
import functools

N_ROWS = 1_000_000
N_COLS = 64
N_IDX = 16_384
NW = 32
R = 31_248
TAIL = N_ROWS - NW * R
R_MAX = R + TAIL
RPAD = ((R_MAX + 1) + 15) // 16 * 16
NCHUNK = N_IDX // 16
WTCH = RPAD // 16
LIST_CAP = N_IDX + 16
CR = 72
NCH = R // CR
WB = 640
PF = 16


@functools.partial(
    pl.kernel,
    out_type=jax.ShapeDtypeStruct((N_ROWS, N_COLS), jnp.float32),
    mesh=plsc.VectorSubcoreMesh(core_axis_name="c", subcore_axis_name="s"),
    compiler_params=pltpu.CompilerParams(needs_layout_passes=False),
    scratch_types=[
        pltpu.VMEM((N_IDX,), jnp.int32),
        pltpu.VMEM((RPAD,), jnp.int32),
        pltpu.VMEM((LIST_CAP,), jnp.int32),
        pltpu.VMEM(((WB + 16) * N_COLS,), jnp.float32),
        pltpu.VMEM((2, CR, N_COLS), jnp.float32),
        pltpu.VMEM((8, N_COLS), jnp.float32),
        pltpu.SemaphoreType.DMA,
        pltpu.SemaphoreType.DMA,
        pltpu.SemaphoreType.DMA,
    ],
)
def _scatter_nd_sc(data_ref, idx_ref, upd1_ref, out_ref,
                   idx_v, win, dst1d, wrow1, cbuf, tbuf, fsem, wsem, pfsem):
    wid = lax.axis_index("s") * 2 + lax.axis_index("c")
    start = pl.multiple_of(wid * R, 8)
    is_last = wid == NW - 1
    mylen = jnp.where(is_last, R_MAX, R)
    iota = lax.iota(jnp.int32, 16)

    pltpu.sync_copy(idx_ref, idx_v)

    neg1 = jnp.full((16,), -1, jnp.int32)

    def ms_body(t, c):
        win[pl.ds(t * 16, 16)] = neg1
        return c

    lax.fori_loop(0, WTCH, ms_body, 0)

    def chunk_body(c, carry):
        v = idx_v[pl.ds(c * 16, 16)]
        rel = v - start
        m = (rel >= 0) & (rel < mylen)
        sel = jnp.where(m, rel, R_MAX)
        j = iota + c * 16
        plsc.store_scatter(win, [sel], j)

        def wbody(_):
            rb = plsc.load_gather(win, [sel])
            need = rb < j
            plsc.store_scatter(win, [sel], j, mask=need)
            return jnp.max(need.astype(jnp.int32)) > 0

        lax.while_loop(lambda keep: keep, wbody, jnp.bool_(True))
        return carry

    lax.fori_loop(0, NCHUNK, chunk_body, 0)

    def ext_body(t, acc):
        w = win[pl.ds(t * 16, 16)]
        slot = iota + t * 16
        m = (w >= 0) & (slot < mylen)
        plsc.store_compressed(dst1d.at[pl.ds(acc, 16)], slot + start, mask=m)
        return acc + jnp.sum(m.astype(jnp.int32))

    acc = lax.fori_loop(0, WTCH, ext_body, jnp.int32(0))

    def prefetch_rows(b0, n):
        ngrp = (n + PF - 1) // PF

        def grp_body(g, c):
            base = b0 + g * PF
            for t in range(PF):
                kk = g * PF + t
                li = jnp.minimum(base + t, acc - 1)
                d = dst1d[pl.ds(li, 16)][0]
                j = win[pl.ds(d - start, 16)][0]
                pltpu.async_copy(
                    upd1_ref.at[pl.ds(j * N_COLS, N_COLS)],
                    wrow1.at[pl.ds(kk * N_COLS, N_COLS)], pfsem)
            for t in range(PF):
                kk = g * PF + t
                pltpu.make_async_copy(
                    upd1_ref.at[pl.ds(0, N_COLS)],
                    wrow1.at[pl.ds(kk * N_COLS, N_COLS)], pfsem).wait()
            return c

        lax.fori_loop(0, ngrp, grp_body, 0)

    wbcap = jnp.minimum(acc, WB)
    prefetch_rows(jnp.int32(0), wbcap)

    def blend(row0, crows, slot, cur0):
        def cond(cur):
            d = dst1d[pl.ds(cur, 16)][0]
            return (cur < wbcap) & (d < row0 + crows)

        def body(cur):
            d = dst1d[pl.ds(cur, 16)][0]
            off = d - row0
            for l in range(N_COLS // 16):
                cbuf[slot, off, pl.ds(l * 16, 16)] = \
                    wrow1[pl.ds(cur * N_COLS + l * 16, 16)]
            return cur + 1

        return lax.while_loop(cond, body, cur0)

    pltpu.async_copy(data_ref.at[pl.ds(start, CR)], cbuf.at[0], fsem)

    def copy_chunk(i, carry):
        s = lax.bitwise_and(i, 1)
        row0 = pl.multiple_of(start + i * CR, 8)
        pltpu.make_async_copy(
            data_ref.at[pl.ds(row0, CR)], cbuf.at[s], fsem).wait()

        @pl.when(i >= 1)
        def _():
            row0m = pl.multiple_of(row0 - CR, 8)
            pltpu.make_async_copy(
                cbuf.at[1 - s], out_ref.at[pl.ds(row0m, CR)], wsem).wait()

        @pl.when(i + 1 < NCH)
        def _():
            row0n = pl.multiple_of(row0 + CR, 8)
            pltpu.async_copy(
                data_ref.at[pl.ds(row0n, CR)], cbuf.at[1 - s], fsem)

        carry = blend(row0, CR, s, carry)
        pltpu.async_copy(cbuf.at[s], out_ref.at[pl.ds(row0, CR)], wsem)
        return carry

    carry = lax.fori_loop(0, NCH, copy_chunk, jnp.int32(0))

    lastrow = pl.multiple_of(start + (NCH - 1) * CR, 8)
    pltpu.make_async_copy(
        cbuf.at[(NCH - 1) % 2], out_ref.at[pl.ds(lastrow, CR)], wsem).wait()

    trow = NW * R
    pltpu.sync_copy(data_ref.at[pl.ds(trow, TAIL)], cbuf.at[0, pl.ds(0, TAIL)])

    @pl.when(is_last)
    def _():
        pltpu.sync_copy(cbuf.at[0, pl.ds(0, TAIL)],
                        out_ref.at[pl.ds(trow, TAIL)])

    def fb_body(k, c):
        d = dst1d[pl.ds(k, 16)][0]
        j = win[pl.ds(d - start, 16)][0]
        g8 = pl.multiple_of((d >> 3) * 8, 8)
        pltpu.sync_copy(out_ref.at[pl.ds(g8, 8)], tbuf)
        pltpu.sync_copy(upd1_ref.at[pl.ds(j * N_COLS, N_COLS)], tbuf.at[d - g8])
        pltpu.sync_copy(tbuf, out_ref.at[pl.ds(g8, 8)])
        return c

    lax.fori_loop(carry, jnp.maximum(acc, carry), fb_body, 0)


def kernel(data, indices, updates):
    idx = indices.reshape(-1).astype(jnp.int32)
    upd1 = updates.reshape(-1)
    return _scatter_nd_sc(data, idx, upd1)


# speedup vs baseline: 1.1615x; 1.1615x over previous
import functools

import jax
import jax.numpy as jnp
from jax import lax
from jax.experimental import pallas as pl
from jax.experimental.pallas import tpu as pltpu
from jax.experimental.pallas import tpu_sc as plsc

N_ROWS = 1_000_000
N_COLS = 64
N_IDX = 16_384
NW = 32
R = 31_248
TAIL = N_ROWS - NW * R
R_MAX = R + TAIL
RPAD = ((R_MAX + 1) + 15) // 16 * 16
WTCH = RPAD // 16
LIST_CAP = N_IDX + 16
IP = 1024
NIP = N_IDX // IP
CR = 72
NCH = R // CR
WB = 608
PF = 16


@functools.partial(
    pl.kernel,
    out_type=jax.ShapeDtypeStruct((N_ROWS, N_COLS), jnp.float32),
    mesh=plsc.VectorSubcoreMesh(core_axis_name="c", subcore_axis_name="s"),
    compiler_params=pltpu.CompilerParams(needs_layout_passes=False),
    scratch_types=[
        pltpu.VMEM((2, IP), jnp.int32),
        pltpu.VMEM((RPAD,), jnp.int32),
        pltpu.VMEM((LIST_CAP,), jnp.int32),
        pltpu.VMEM(((WB + 16) * N_COLS,), jnp.float32),
        pltpu.VMEM((4, CR, N_COLS), jnp.float32),
        pltpu.VMEM((8, N_COLS), jnp.float32),
        pltpu.SemaphoreType.DMA,
        pltpu.SemaphoreType.DMA,
        pltpu.SemaphoreType.DMA,
        pltpu.SemaphoreType.DMA,
        pltpu.SemaphoreType.DMA,
        pltpu.SemaphoreType.DMA,
    ],
)
def _scatter_nd_sc(data_ref, idx_ref, upd1_ref, out_ref,
                   ibuf, win, dst1d, wrow1, cbuf, tbuf,
                   isem, fsem_a, fsem_b, wsem_a, wsem_b, pfsem):
    wid = lax.axis_index("s") * 2 + lax.axis_index("c")
    start = pl.multiple_of(wid * R, 8)
    is_last = wid == NW - 1
    mylen = jnp.where(is_last, R_MAX, R)
    iota = lax.iota(jnp.int32, 16)

    neg1 = jnp.full((16,), -1, jnp.int32)

    def ms_body(t, c):
        win[pl.ds(t * 16, 16)] = neg1
        return c

    lax.fori_loop(0, WTCH, ms_body, 0)

    pltpu.async_copy(idx_ref.at[pl.ds(0, IP)], ibuf.at[0], isem)

    def piece_body(p, carry):
        sp = lax.bitwise_and(p, 1)
        pltpu.make_async_copy(
            idx_ref.at[pl.ds(p * IP, IP)], ibuf.at[sp], isem).wait()

        @pl.when(p + 1 < NIP)
        def _():
            pltpu.async_copy(
                idx_ref.at[pl.ds((p + 1) * IP, IP)], ibuf.at[1 - sp], isem)

        def chunk_body(c, carry2):
            v = ibuf[sp, pl.ds(c * 16, 16)]
            rel = v - start
            m = (rel >= 0) & (rel < mylen)
            sel = jnp.where(m, rel, R_MAX)
            j = iota + (p * IP + c * 16)
            plsc.store_scatter(win, [sel], j)

            def wbody(_):
                rb = plsc.load_gather(win, [sel])
                need = rb < j
                plsc.store_scatter(win, [sel], j, mask=need)
                return jnp.max(need.astype(jnp.int32)) > 0

            lax.while_loop(lambda keep: keep, wbody, jnp.bool_(True))
            return carry2

        lax.fori_loop(0, IP // 16, chunk_body, 0)
        return carry

    lax.fori_loop(0, NIP, piece_body, 0)

    def ext_body(t, acc):
        w = win[pl.ds(t * 16, 16)]
        slot = iota + t * 16
        m = (w >= 0) & (slot < mylen)
        plsc.store_compressed(dst1d.at[pl.ds(acc, 16)], slot + start, mask=m)
        return acc + jnp.sum(m.astype(jnp.int32))

    acc = lax.fori_loop(0, WTCH, ext_body, jnp.int32(0))

    def prefetch_rows(b0, n):
        ngrp = (n + PF - 1) // PF

        def grp_body(g, c):
            base = b0 + g * PF
            for t in range(PF):
                kk = g * PF + t
                li = jnp.minimum(base + t, acc - 1)
                d = dst1d[pl.ds(li, 16)][0]
                j = win[pl.ds(d - start, 16)][0]
                pltpu.async_copy(
                    upd1_ref.at[pl.ds(j * N_COLS, N_COLS)],
                    wrow1.at[pl.ds(kk * N_COLS, N_COLS)], pfsem)
            for t in range(PF):
                kk = g * PF + t
                pltpu.make_async_copy(
                    upd1_ref.at[pl.ds(0, N_COLS)],
                    wrow1.at[pl.ds(kk * N_COLS, N_COLS)], pfsem).wait()
            return c

        lax.fori_loop(0, ngrp, grp_body, 0)

    wbcap = jnp.minimum(acc, WB)
    prefetch_rows(jnp.int32(0), wbcap)

    def blend(row0, crows, slot, cur0):
        def cond(cur):
            d = dst1d[pl.ds(cur, 16)][0]
            return (cur < wbcap) & (d < row0 + crows)

        def body(cur):
            d = dst1d[pl.ds(cur, 16)][0]
            off = d - row0
            for l in range(N_COLS // 16):
                cbuf[slot, off, pl.ds(l * 16, 16)] = \
                    wrow1[pl.ds(cur * N_COLS + l * 16, 16)]
            return cur + 1

        return lax.while_loop(cond, body, cur0)

    def rowof(i):
        return pl.multiple_of(start + i * CR, 8)

    pltpu.async_copy(data_ref.at[pl.ds(rowof(0), CR)], cbuf.at[0], fsem_a)
    pltpu.async_copy(data_ref.at[pl.ds(rowof(1), CR)], cbuf.at[1], fsem_b)

    def do_chunk(i, cur, fsem, wsem):
        sl = lax.bitwise_and(i, 3)
        sl2 = lax.bitwise_and(i + 2, 3)
        pltpu.make_async_copy(
            data_ref.at[pl.ds(rowof(i), CR)], cbuf.at[sl], fsem).wait()

        @pl.when(i >= 2)
        def _():
            pltpu.make_async_copy(
                cbuf.at[sl2], out_ref.at[pl.ds(rowof(i - 2), CR)], wsem).wait()

        @pl.when(i + 2 < NCH)
        def _():
            pltpu.async_copy(
                data_ref.at[pl.ds(rowof(i + 2), CR)], cbuf.at[sl2], fsem)

        cur = blend(rowof(i), CR, sl, cur)
        pltpu.async_copy(cbuf.at[sl], out_ref.at[pl.ds(rowof(i), CR)], wsem)
        return cur

    def pair_body(i2, cur):
        cur = do_chunk(i2 * 2, cur, fsem_a, wsem_a)
        cur = do_chunk(i2 * 2 + 1, cur, fsem_b, wsem_b)
        return cur

    cur = lax.fori_loop(0, NCH // 2, pair_body, jnp.int32(0))

    pltpu.make_async_copy(
        cbuf.at[(NCH - 2) % 4], out_ref.at[pl.ds(rowof(NCH - 2), CR)],
        wsem_a).wait()
    pltpu.make_async_copy(
        cbuf.at[(NCH - 1) % 4], out_ref.at[pl.ds(rowof(NCH - 1), CR)],
        wsem_b).wait()

    trow = NW * R

    @pl.when(is_last)
    def _():
        pltpu.sync_copy(data_ref.at[pl.ds(trow, TAIL)],
                        cbuf.at[0, pl.ds(0, TAIL)])
        pltpu.sync_copy(cbuf.at[0, pl.ds(0, TAIL)],
                        out_ref.at[pl.ds(trow, TAIL)])

    def fb_body(k, c):
        d = dst1d[pl.ds(k, 16)][0]
        j = win[pl.ds(d - start, 16)][0]
        g8 = pl.multiple_of((d >> 3) * 8, 8)
        pltpu.sync_copy(out_ref.at[pl.ds(g8, 8)], tbuf)
        pltpu.sync_copy(upd1_ref.at[pl.ds(j * N_COLS, N_COLS)], tbuf.at[d - g8])
        pltpu.sync_copy(tbuf, out_ref.at[pl.ds(g8, 8)])
        return c

    lax.fori_loop(cur, jnp.maximum(acc, cur), fb_body, 0)


def kernel(data, indices, updates):
    idx = indices.reshape(-1).astype(jnp.int32)
    upd1 = updates.reshape(-1)
    return _scatter_nd_sc(data, idx, upd1)


# speedup vs baseline: 4.2264x; 3.6387x over previous
import functools

import jax
import jax.numpy as jnp
from jax import lax
from jax.experimental import pallas as pl
from jax.experimental.pallas import tpu as pltpu
from jax.experimental.pallas import tpu_sc as plsc

N_ROWS = 1_000_000
N_COLS = 64
N_IDX = 16_384
NW = 32
R = 31_232
TRIM = 999_936
SL_MAX = TRIM - (NW - 1) * R
RPAD = ((SL_MAX + 1) + 15) // 16 * 16
WTCH = RPAD // 16
LIST_CAP = N_IDX + 16
IP = 1024
NIP = N_IDX // IP
CR = 128
NCH = R // CR
WB = 576
PF = 16


@functools.partial(
    pl.kernel,
    out_type=jax.ShapeDtypeStruct((N_COLS, N_ROWS), jnp.float32),
    mesh=plsc.VectorSubcoreMesh(core_axis_name="c", subcore_axis_name="s"),
    compiler_params=pltpu.CompilerParams(needs_layout_passes=False),
    scratch_types=[
        pltpu.VMEM((2, IP), jnp.int32),
        pltpu.VMEM((RPAD,), jnp.int32),
        pltpu.VMEM((LIST_CAP,), jnp.int32),
        pltpu.VMEM((4, N_COLS, CR), jnp.float32),
        pltpu.VMEM((N_COLS, CR), jnp.float32),
        pltpu.VMEM(((WB + 16) * N_COLS,), jnp.float32),
        pltpu.SemaphoreType.DMA,
        pltpu.SemaphoreType.DMA,
        pltpu.SemaphoreType.DMA,
        pltpu.SemaphoreType.DMA,
        pltpu.SemaphoreType.DMA,
        pltpu.SemaphoreType.DMA,
    ],
)
def _scatter_nd_sc(dataT_ref, idx_ref, upd1_ref, outT_ref,
                   ibuf, win, dst1d, cbuf, tbuf, wrow1,
                   isem, fsem_a, fsem_b, wsem_a, wsem_b, pfsem):
    sid = lax.axis_index("s")
    wid = sid * 2 + lax.axis_index("c")
    start = pl.multiple_of(wid * R, 128)
    is_last = wid == NW - 1
    mylen = jnp.where(is_last, SL_MAX, R)
    nch = jnp.where(is_last, NCH + (SL_MAX - R) // CR, NCH)
    iota = lax.iota(jnp.int32, 16)

    neg1 = jnp.full((16,), -1, jnp.int32)

    def ms_body(t, c):
        win[pl.ds(t * 16, 16)] = neg1
        return c

    lax.fori_loop(0, WTCH, ms_body, 0)

    pltpu.async_copy(idx_ref.at[pl.ds(0, IP)], ibuf.at[0], isem)

    def piece_body(p, carry):
        sp = lax.bitwise_and(p, 1)
        pltpu.make_async_copy(
            idx_ref.at[pl.ds(p * IP, IP)], ibuf.at[sp], isem).wait()

        @pl.when(p + 1 < NIP)
        def _():
            pltpu.async_copy(
                idx_ref.at[pl.ds((p + 1) * IP, IP)], ibuf.at[1 - sp], isem)

        def chunk_body(c, carry2):
            v = ibuf[sp, pl.ds(c * 16, 16)]
            rel = v - start
            m = (rel >= 0) & (rel < mylen)
            sel = jnp.where(m, rel, SL_MAX)
            j = iota + (p * IP + c * 16)
            plsc.store_scatter(win, [sel], j)

            def wbody(_):
                rb = plsc.load_gather(win, [sel])
                need = rb < j
                plsc.store_scatter(win, [sel], j, mask=need)
                return jnp.max(need.astype(jnp.int32)) > 0

            lax.while_loop(lambda keep: keep, wbody, jnp.bool_(True))
            return carry2

        lax.fori_loop(0, IP // 16, chunk_body, 0)
        return carry

    lax.fori_loop(0, NIP, piece_body, 0)

    def ext_body(t, acc):
        w = win[pl.ds(t * 16, 16)]
        slot = iota + t * 16
        m = (w >= 0) & (slot < mylen)
        plsc.store_compressed(dst1d.at[pl.ds(acc, 16)], slot + start, mask=m)
        return acc + jnp.sum(m.astype(jnp.int32))

    acc = lax.fori_loop(0, WTCH, ext_body, jnp.int32(0))

    def prefetch_rows(b0, n):
        ngrp = (n + PF - 1) // PF

        def grp_body(g, c):
            base = b0 + g * PF
            for t in range(PF):
                kk = g * PF + t
                li = jnp.minimum(base + t, acc - 1)
                d = dst1d[pl.ds(li, 16)][0]
                j = win[pl.ds(d - start, 16)][0]
                pltpu.async_copy(
                    upd1_ref.at[pl.ds(j * N_COLS, N_COLS)],
                    wrow1.at[pl.ds(kk * N_COLS, N_COLS)], pfsem)
            for t in range(PF):
                kk = g * PF + t
                pltpu.make_async_copy(
                    upd1_ref.at[pl.ds(0, N_COLS)],
                    wrow1.at[pl.ds(kk * N_COLS, N_COLS)], pfsem).wait()
            return c

        lax.fori_loop(0, ngrp, grp_body, 0)

    wbcap = jnp.minimum(acc, WB)
    prefetch_rows(jnp.int32(0), wbcap)

    def splice(ref2d, off, cur):
        for l in range(N_COLS // 16):
            plsc.store_scatter(
                ref2d,
                [iota + l * 16, jnp.full((16,), 0, jnp.int32) + off],
                wrow1[pl.ds(cur * N_COLS + l * 16, 16)])

    def blend(row0, crows, slot, cur0):
        def cond(cur):
            d = dst1d[pl.ds(cur, 16)][0]
            return (cur < wbcap) & (d < row0 + crows)

        def body(cur):
            d = dst1d[pl.ds(cur, 16)][0]
            splice(cbuf.at[slot], d - row0, cur)
            return cur + 1

        return lax.while_loop(cond, body, cur0)

    def rowof(i):
        return pl.multiple_of(start + i * CR, 128)

    pltpu.async_copy(dataT_ref.at[:, pl.ds(rowof(0), CR)], cbuf.at[0], fsem_a)
    pltpu.async_copy(dataT_ref.at[:, pl.ds(rowof(1), CR)], cbuf.at[1], fsem_b)

    def do_chunk(i, cur, fsem, wsem):
        sl = lax.bitwise_and(i, 3)
        sl2 = lax.bitwise_and(i + 2, 3)
        pltpu.make_async_copy(
            dataT_ref.at[:, pl.ds(rowof(i), CR)], cbuf.at[sl], fsem).wait()

        @pl.when(i >= 2)
        def _():
            pltpu.make_async_copy(
                cbuf.at[sl2], outT_ref.at[:, pl.ds(rowof(i - 2), CR)],
                wsem).wait()

        @pl.when(i + 2 < nch)
        def _():
            pltpu.async_copy(
                dataT_ref.at[:, pl.ds(rowof(i + 2), CR)], cbuf.at[sl2], fsem)

        cur = blend(rowof(i), CR, sl, cur)
        pltpu.async_copy(cbuf.at[sl], outT_ref.at[:, pl.ds(rowof(i), CR)], wsem)
        return cur

    def pair_body(i2, cur):
        cur = do_chunk(i2 * 2, cur, fsem_a, wsem_a)
        cur = do_chunk(i2 * 2 + 1, cur, fsem_b, wsem_b)
        return cur

    cur = lax.fori_loop(0, nch // 2, pair_body, jnp.int32(0))

    pltpu.make_async_copy(
        cbuf.at[lax.bitwise_and(nch - 2, 3)],
        outT_ref.at[:, pl.ds(rowof(nch - 2), CR)], wsem_a).wait()
    pltpu.make_async_copy(
        cbuf.at[lax.bitwise_and(nch - 1, 3)],
        outT_ref.at[:, pl.ds(rowof(nch - 1), CR)], wsem_b).wait()

    def fb_body(k, c):
        d = dst1d[pl.ds(k, 16)][0]
        j = win[pl.ds(d - start, 16)][0]
        t0 = pl.multiple_of((d >> 7) * CR, 128)
        pltpu.sync_copy(upd1_ref.at[pl.ds(j * N_COLS, N_COLS)],
                        wrow1.at[pl.ds(WB * N_COLS, N_COLS)])
        pltpu.sync_copy(outT_ref.at[:, pl.ds(t0, CR)], tbuf)
        splice(tbuf, d - t0, jnp.int32(WB))
        pltpu.sync_copy(tbuf, outT_ref.at[:, pl.ds(t0, CR)])
        return c

    lax.fori_loop(cur, jnp.maximum(acc, cur), fb_body, 0)


def kernel(data, indices, updates):
    idx = indices.reshape(-1).astype(jnp.int32)
    upd1 = updates.reshape(-1)
    outT = _scatter_nd_sc(data.T, idx, upd1)
    tail = data[TRIM:, :]
    tidx = jnp.where(idx >= TRIM, idx - TRIM, N_ROWS - TRIM)
    tail = tail.at[tidx].set(updates)
    outT = lax.dynamic_update_slice(outT, tail.T, (0, TRIM))
    return outT.T


# speedup vs baseline: 4.3343x; 1.0255x over previous
import functools

import jax
import jax.numpy as jnp
from jax import lax
from jax.experimental import pallas as pl
from jax.experimental.pallas import tpu as pltpu
from jax.experimental.pallas import tpu_sc as plsc

N_ROWS = 1_000_000
N_COLS = 64
N_IDX = 16_384
NW = 32
R = 31_232
TRIM = 999_936
SL_MAX = TRIM - (NW - 1) * R
RPAD = ((SL_MAX + 1) + 15) // 16 * 16
WTCH = RPAD // 16
LIST_CAP = N_IDX + 16
IP = 1024
NIP = N_IDX // IP
CR = 128
NCH = R // CR
WB = 576
PF = 16


@functools.partial(
    pl.kernel,
    out_type=jax.ShapeDtypeStruct((N_COLS, N_ROWS), jnp.float32),
    mesh=plsc.VectorSubcoreMesh(core_axis_name="c", subcore_axis_name="s"),
    compiler_params=pltpu.CompilerParams(needs_layout_passes=False),
    scratch_types=[
        pltpu.VMEM((2, IP), jnp.int32),
        pltpu.VMEM((RPAD,), jnp.int32),
        pltpu.VMEM((LIST_CAP,), jnp.int32),
        pltpu.VMEM((4, N_COLS, CR), jnp.float32),
        pltpu.VMEM((N_COLS, CR), jnp.float32),
        pltpu.VMEM(((WB + 16) * N_COLS,), jnp.float32),
        pltpu.SemaphoreType.DMA,
        pltpu.SemaphoreType.DMA,
        pltpu.SemaphoreType.DMA,
        pltpu.SemaphoreType.DMA,
        pltpu.SemaphoreType.DMA,
        pltpu.SemaphoreType.DMA,
    ],
)
def _scatter_nd_sc(dataT_ref, idx_ref, upd1_ref, outT_ref,
                   ibuf, win, dst1d, cbuf, tbuf, wrow1,
                   isem, fsem_a, fsem_b, wsem_a, wsem_b, pfsem):
    sid = lax.axis_index("s")
    wid = sid * 2 + lax.axis_index("c")
    start = pl.multiple_of(wid * R, 128)
    is_last = wid == NW - 1
    mylen = jnp.where(is_last, SL_MAX, R)
    nch = jnp.where(is_last, NCH + (SL_MAX - R) // CR, NCH)
    iota = lax.iota(jnp.int32, 16)

    neg1 = jnp.full((16,), -1, jnp.int32)

    def ms_body(t, c):
        win[pl.ds(t * 16, 16)] = neg1
        return c

    lax.fori_loop(0, WTCH, ms_body, 0, unroll=8)

    pltpu.async_copy(idx_ref.at[pl.ds(0, IP)], ibuf.at[0], isem)

    def piece_body(p, carry):
        sp = lax.bitwise_and(p, 1)
        pltpu.make_async_copy(
            idx_ref.at[pl.ds(p * IP, IP)], ibuf.at[sp], isem).wait()

        @pl.when(p + 1 < NIP)
        def _():
            pltpu.async_copy(
                idx_ref.at[pl.ds((p + 1) * IP, IP)], ibuf.at[1 - sp], isem)

        def chunk_body(c, carry2):
            v = ibuf[sp, pl.ds(c * 16, 16)]
            rel = v - start
            m = (rel >= 0) & (rel < mylen)
            sel = jnp.where(m, rel, SL_MAX)
            j = iota + (p * IP + c * 16)
            plsc.store_scatter(win, [sel], j)

            def wbody(_):
                rb = plsc.load_gather(win, [sel])
                need = rb < j
                plsc.store_scatter(win, [sel], j, mask=need)
                return jnp.max(need.astype(jnp.int32)) > 0

            lax.while_loop(lambda keep: keep, wbody, jnp.bool_(True))
            return carry2

        lax.fori_loop(0, IP // 16, chunk_body, 0, unroll=2)
        return carry

    lax.fori_loop(0, NIP, piece_body, 0)

    def ext_body(t, acc):
        w = win[pl.ds(t * 16, 16)]
        slot = iota + t * 16
        m = (w >= 0) & (slot < mylen)
        plsc.store_compressed(dst1d.at[pl.ds(acc, 16)], slot + start, mask=m)
        return acc + jnp.sum(m.astype(jnp.int32))

    acc = lax.fori_loop(0, WTCH, ext_body, jnp.int32(0), unroll=4)

    def prefetch_rows(b0, n):
        ngrp = (n + PF - 1) // PF

        def grp_body(g, c):
            base = b0 + g * PF
            for t in range(PF):
                kk = g * PF + t
                li = jnp.minimum(base + t, acc - 1)
                d = dst1d[pl.ds(li, 16)][0]
                j = win[pl.ds(d - start, 16)][0]
                pltpu.async_copy(
                    upd1_ref.at[pl.ds(j * N_COLS, N_COLS)],
                    wrow1.at[pl.ds(kk * N_COLS, N_COLS)], pfsem)
            for t in range(PF):
                kk = g * PF + t
                pltpu.make_async_copy(
                    upd1_ref.at[pl.ds(0, N_COLS)],
                    wrow1.at[pl.ds(kk * N_COLS, N_COLS)], pfsem).wait()
            return c

        lax.fori_loop(0, ngrp, grp_body, 0)

    wbcap = jnp.minimum(acc, WB)
    prefetch_rows(jnp.int32(0), wbcap)

    def splice(ref2d, off, cur):
        for l in range(N_COLS // 16):
            plsc.store_scatter(
                ref2d,
                [iota + l * 16, jnp.full((16,), 0, jnp.int32) + off],
                wrow1[pl.ds(cur * N_COLS + l * 16, 16)])

    def blend(row0, crows, slot, cur0):
        def cond(cur):
            d = dst1d[pl.ds(cur, 16)][0]
            return (cur < wbcap) & (d < row0 + crows)

        def body(cur):
            d = dst1d[pl.ds(cur, 16)][0]
            splice(cbuf.at[slot], d - row0, cur)
            return cur + 1

        return lax.while_loop(cond, body, cur0)

    def rowof(i):
        return pl.multiple_of(start + i * CR, 128)

    pltpu.async_copy(dataT_ref.at[:, pl.ds(rowof(0), CR)], cbuf.at[0], fsem_a)
    pltpu.async_copy(dataT_ref.at[:, pl.ds(rowof(1), CR)], cbuf.at[1], fsem_b)

    def do_chunk(i, cur, fsem, wsem):
        sl = lax.bitwise_and(i, 3)
        sl2 = lax.bitwise_and(i + 2, 3)
        pltpu.make_async_copy(
            dataT_ref.at[:, pl.ds(rowof(i), CR)], cbuf.at[sl], fsem).wait()

        @pl.when(i >= 2)
        def _():
            pltpu.make_async_copy(
                cbuf.at[sl2], outT_ref.at[:, pl.ds(rowof(i - 2), CR)],
                wsem).wait()

        @pl.when(i + 2 < nch)
        def _():
            pltpu.async_copy(
                dataT_ref.at[:, pl.ds(rowof(i + 2), CR)], cbuf.at[sl2], fsem)

        cur = blend(rowof(i), CR, sl, cur)
        pltpu.async_copy(cbuf.at[sl], outT_ref.at[:, pl.ds(rowof(i), CR)], wsem)
        return cur

    def pair_body(i2, cur):
        cur = do_chunk(i2 * 2, cur, fsem_a, wsem_a)
        cur = do_chunk(i2 * 2 + 1, cur, fsem_b, wsem_b)
        return cur

    cur = lax.fori_loop(0, nch // 2, pair_body, jnp.int32(0))

    pltpu.make_async_copy(
        cbuf.at[lax.bitwise_and(nch - 2, 3)],
        outT_ref.at[:, pl.ds(rowof(nch - 2), CR)], wsem_a).wait()
    pltpu.make_async_copy(
        cbuf.at[lax.bitwise_and(nch - 1, 3)],
        outT_ref.at[:, pl.ds(rowof(nch - 1), CR)], wsem_b).wait()

    def fb_body(k, c):
        d = dst1d[pl.ds(k, 16)][0]
        j = win[pl.ds(d - start, 16)][0]
        t0 = pl.multiple_of((d >> 7) * CR, 128)
        pltpu.sync_copy(upd1_ref.at[pl.ds(j * N_COLS, N_COLS)],
                        wrow1.at[pl.ds(WB * N_COLS, N_COLS)])
        pltpu.sync_copy(outT_ref.at[:, pl.ds(t0, CR)], tbuf)
        splice(tbuf, d - t0, jnp.int32(WB))
        pltpu.sync_copy(tbuf, outT_ref.at[:, pl.ds(t0, CR)])
        return c

    lax.fori_loop(cur, jnp.maximum(acc, cur), fb_body, 0)


def kernel(data, indices, updates):
    idx = indices.reshape(-1).astype(jnp.int32)
    upd1 = updates.reshape(-1)
    outT = _scatter_nd_sc(data.T, idx, upd1)
    tail = data[TRIM:, :]
    tidx = jnp.where(idx >= TRIM, idx - TRIM, N_ROWS - TRIM)
    tail = tail.at[tidx].set(updates)
    outT = lax.dynamic_update_slice(outT, tail.T, (0, TRIM))
    return outT.T


# speedup vs baseline: 4.4348x; 1.0232x over previous
import functools

import jax
import jax.numpy as jnp
from jax import lax
from jax.experimental import pallas as pl
from jax.experimental.pallas import tpu as pltpu
from jax.experimental.pallas import tpu_sc as plsc

N_ROWS = 1_000_000
N_COLS = 64
N_IDX = 16_384
NW = 32
R = 31_232
TRIM = 999_936
SL_MAX = TRIM - (NW - 1) * R
RPAD = ((SL_MAX + 1) + 15) // 16 * 16
WTCH = RPAD // 16
LIST_CAP = N_IDX + 16
IP = 1024
NIP = N_IDX // IP
CR = 128
NCH = R // CR
WB = 576
PF = 16


@functools.partial(
    pl.kernel,
    out_type=jax.ShapeDtypeStruct((N_COLS, N_ROWS), jnp.float32),
    mesh=plsc.VectorSubcoreMesh(core_axis_name="c", subcore_axis_name="s"),
    compiler_params=pltpu.CompilerParams(needs_layout_passes=False),
    scratch_types=[
        pltpu.VMEM((2, IP), jnp.int32),
        pltpu.VMEM((RPAD,), jnp.int32),
        pltpu.VMEM((LIST_CAP,), jnp.int32),
        pltpu.VMEM((4, N_COLS, CR), jnp.float32),
        pltpu.VMEM((N_COLS, CR), jnp.float32),
        pltpu.VMEM(((WB + 16) * N_COLS,), jnp.float32),
        pltpu.SemaphoreType.DMA,
        pltpu.SemaphoreType.DMA,
        pltpu.SemaphoreType.DMA,
        pltpu.SemaphoreType.DMA,
        pltpu.SemaphoreType.DMA,
        pltpu.SemaphoreType.DMA,
    ],
)
def _scatter_nd_sc(dataT_ref, idx_ref, upd1_ref, outT_ref,
                   ibuf, win, dst1d, cbuf, tbuf, wrow1,
                   isem, fsem_a, fsem_b, wsem_a, wsem_b, pfsem):
    sid = lax.axis_index("s")
    wid = sid * 2 + lax.axis_index("c")
    start = pl.multiple_of(wid * R, 128)
    is_last = wid == NW - 1
    mylen = jnp.where(is_last, SL_MAX, R)
    nch = jnp.where(is_last, NCH + (SL_MAX - R) // CR, NCH)
    iota = lax.iota(jnp.int32, 16)

    neg1 = jnp.full((16,), -1, jnp.int32)

    def ms_body(t, c):
        win[pl.ds(t * 16, 16)] = neg1
        return c

    lax.fori_loop(0, WTCH, ms_body, 0, unroll=8)

    pltpu.async_copy(idx_ref.at[pl.ds(0, IP)], ibuf.at[0], isem)

    def piece_body(p, carry):
        sp = lax.bitwise_and(p, 1)
        pltpu.make_async_copy(
            idx_ref.at[pl.ds(p * IP, IP)], ibuf.at[sp], isem).wait()

        @pl.when(p + 1 < NIP)
        def _():
            pltpu.async_copy(
                idx_ref.at[pl.ds((p + 1) * IP, IP)], ibuf.at[1 - sp], isem)

        def chunk_body(c, carry2):
            v = ibuf[sp, pl.ds(c * 16, 16)]
            rel = v - start
            m = (rel >= 0) & (rel < mylen)
            sel = jnp.where(m, rel, SL_MAX)
            j = iota + (p * IP + c * 16)
            plsc.store_scatter(win, [sel], j)

            def wbody(_):
                rb = plsc.load_gather(win, [sel])
                need = rb < j
                plsc.store_scatter(win, [sel], j, mask=need)
                return plsc.all_reduce_population_count(need)[0] > 0

            lax.while_loop(lambda keep: keep, wbody, jnp.bool_(True))
            return carry2

        lax.fori_loop(0, IP // 16, chunk_body, 0, unroll=2)
        return carry

    lax.fori_loop(0, NIP, piece_body, 0)

    def ext_body(t, acc):
        w = win[pl.ds(t * 16, 16)]
        slot = iota + t * 16
        m = (w >= 0) & (slot < mylen)
        plsc.store_compressed(dst1d.at[pl.ds(acc, 16)], slot + start, mask=m)
        return acc + plsc.all_reduce_population_count(m)[0]

    acc = lax.fori_loop(0, WTCH, ext_body, jnp.int32(0), unroll=4)

    def prefetch_rows(b0, n):
        ngrp = (n + PF - 1) // PF

        def grp_body(g, c):
            base = b0 + g * PF
            for t in range(PF):
                kk = g * PF + t
                li = jnp.minimum(base + t, acc - 1)
                d = dst1d[pl.ds(li, 16)][0]
                j = win[pl.ds(d - start, 16)][0]
                pltpu.async_copy(
                    upd1_ref.at[pl.ds(j * N_COLS, N_COLS)],
                    wrow1.at[pl.ds(kk * N_COLS, N_COLS)], pfsem)
            for t in range(PF):
                kk = g * PF + t
                pltpu.make_async_copy(
                    upd1_ref.at[pl.ds(0, N_COLS)],
                    wrow1.at[pl.ds(kk * N_COLS, N_COLS)], pfsem).wait()
            return c

        lax.fori_loop(0, ngrp, grp_body, 0)

    wbcap = jnp.minimum(acc, WB)
    prefetch_rows(jnp.int32(0), wbcap)

    def splice(ref2d, off, cur):
        for l in range(N_COLS // 16):
            plsc.store_scatter(
                ref2d,
                [iota + l * 16, jnp.full((16,), 0, jnp.int32) + off],
                wrow1[pl.ds(cur * N_COLS + l * 16, 16)])

    def blend(row0, crows, slot, cur0):
        def cond(cur):
            d = dst1d[pl.ds(cur, 16)][0]
            return (cur < wbcap) & (d < row0 + crows)

        def body(cur):
            d = dst1d[pl.ds(cur, 16)][0]
            splice(cbuf.at[slot], d - row0, cur)
            return cur + 1

        return lax.while_loop(cond, body, cur0)

    def rowof(i):
        return pl.multiple_of(start + i * CR, 128)

    pltpu.async_copy(dataT_ref.at[:, pl.ds(rowof(0), CR)], cbuf.at[0], fsem_a)
    pltpu.async_copy(dataT_ref.at[:, pl.ds(rowof(1), CR)], cbuf.at[1], fsem_b)

    def do_chunk(i, cur, fsem, wsem):
        sl = lax.bitwise_and(i, 3)
        sl2 = lax.bitwise_and(i + 2, 3)
        pltpu.make_async_copy(
            dataT_ref.at[:, pl.ds(rowof(i), CR)], cbuf.at[sl], fsem).wait()

        @pl.when(i >= 2)
        def _():
            pltpu.make_async_copy(
                cbuf.at[sl2], outT_ref.at[:, pl.ds(rowof(i - 2), CR)],
                wsem).wait()

        @pl.when(i + 2 < nch)
        def _():
            pltpu.async_copy(
                dataT_ref.at[:, pl.ds(rowof(i + 2), CR)], cbuf.at[sl2], fsem)

        cur = blend(rowof(i), CR, sl, cur)
        pltpu.async_copy(cbuf.at[sl], outT_ref.at[:, pl.ds(rowof(i), CR)], wsem)
        return cur

    def pair_body(i2, cur):
        cur = do_chunk(i2 * 2, cur, fsem_a, wsem_a)
        cur = do_chunk(i2 * 2 + 1, cur, fsem_b, wsem_b)
        return cur

    cur = lax.fori_loop(0, nch // 2, pair_body, jnp.int32(0))

    pltpu.make_async_copy(
        cbuf.at[lax.bitwise_and(nch - 2, 3)],
        outT_ref.at[:, pl.ds(rowof(nch - 2), CR)], wsem_a).wait()
    pltpu.make_async_copy(
        cbuf.at[lax.bitwise_and(nch - 1, 3)],
        outT_ref.at[:, pl.ds(rowof(nch - 1), CR)], wsem_b).wait()

    def fb_body(k, c):
        d = dst1d[pl.ds(k, 16)][0]
        j = win[pl.ds(d - start, 16)][0]
        t0 = pl.multiple_of((d >> 7) * CR, 128)
        pltpu.sync_copy(upd1_ref.at[pl.ds(j * N_COLS, N_COLS)],
                        wrow1.at[pl.ds(WB * N_COLS, N_COLS)])
        pltpu.sync_copy(outT_ref.at[:, pl.ds(t0, CR)], tbuf)
        splice(tbuf, d - t0, jnp.int32(WB))
        pltpu.sync_copy(tbuf, outT_ref.at[:, pl.ds(t0, CR)])
        return c

    lax.fori_loop(cur, jnp.maximum(acc, cur), fb_body, 0)


def kernel(data, indices, updates):
    idx = indices.reshape(-1).astype(jnp.int32)
    upd1 = updates.reshape(-1)
    outT = _scatter_nd_sc(data.T, idx, upd1)
    tail = data[TRIM:, :]
    tidx = jnp.where(idx >= TRIM, idx - TRIM, N_ROWS - TRIM)
    tail = tail.at[tidx].set(updates)
    outT = lax.dynamic_update_slice(outT, tail.T, (0, TRIM))
    return outT.T
